# baseline (speedup 1.0000x reference)
from kernel_sc import sc_kernel


def kernel(inputs):
    return sc_kernel(inputs)


# speedup vs baseline: 5.9193x; 5.9193x over previous
import jax
import jax.numpy as jnp
from jax.experimental import pallas as pl
from jax.experimental.pallas import tpu as pltpu

TOPK = 8
ROWS = 512
CBLK = 4096

_SORT8 = [
    (0, 1), (2, 3), (4, 5), (6, 7),
    (0, 2), (1, 3), (4, 6), (5, 7),
    (1, 2), (5, 6),
    (0, 4), (1, 5), (2, 6), (3, 7),
    (2, 4), (3, 5),
    (1, 2), (3, 4), (5, 6),
]

_BITONIC8 = [
    (0, 4), (1, 5), (2, 6), (3, 7),
    (0, 2), (1, 3), (4, 6), (5, 7),
    (0, 1), (2, 3), (4, 5), (6, 7),
]


def _apply_network(vals, network):
    for i, j in network:
        hi = jnp.maximum(vals[i], vals[j])
        lo = jnp.minimum(vals[i], vals[j])
        vals[i], vals[j] = hi, lo
    return vals


def _topk_block(x_ref, o_ref, s_ref):
    R = x_ref.shape[0]
    ncol = x_ref.shape[1]
    neg = jnp.float32(-jnp.inf)

    S = None
    for g in range(ncol // 1024):
        grp = [x_ref[:, 128 * (8 * g + j):128 * (8 * g + j + 1)] for j in range(8)]
        grp = _apply_network(grp, _SORT8)
        if S is None:
            S = grp
        else:
            c = [jnp.maximum(S[i], grp[7 - i]) for i in range(8)]
            S = _apply_network(c, _BITONIC8)

    last = pl.num_programs(1) - 1

    def _merge_with_scratch():
        prev = [s_ref[:, 128 * i:128 * (i + 1)] for i in range(8)]
        c = [jnp.maximum(prev[i], S[7 - i]) for i in range(8)]
        return _apply_network(c, _BITONIC8)

    @pl.when(pl.program_id(1) == 0)
    def _():
        for i in range(8):
            s_ref[:, 128 * i:128 * (i + 1)] = S[i]

    @pl.when((pl.program_id(1) > 0) & (pl.program_id(1) < last))
    def _():
        merged = _merge_with_scratch()
        for i in range(8):
            s_ref[:, 128 * i:128 * (i + 1)] = merged[i]

    @pl.when(pl.program_id(1) == last)
    def _():
        Sp = _merge_with_scratch() if last > 0 else list(S)
        lane_iota = jax.lax.broadcasted_iota(jnp.int32, (R, 128), 1)
        outs = []
        for i in range(TOPK):
            m = jnp.max(Sp[0], axis=1, keepdims=True)
            outs.append(m)
            if i < TOPK - 1:
                li = jnp.where(Sp[0] == m, lane_iota, 128)
                first = jnp.min(li, axis=1, keepdims=True)
                mask = lane_iota == first
                depth = TOPK - 1 - i
                for j in range(depth):
                    Sp[j] = jnp.where(mask, Sp[j + 1], Sp[j])
                Sp[depth] = jnp.where(mask, neg, Sp[depth])

        o_ref[...] = jnp.concatenate(outs, axis=1)


def kernel(inputs):
    B, Sdim, N = inputs.shape
    x = inputs.reshape(B * Sdim, N)
    out = pl.pallas_call(
        _topk_block,
        grid=((B * Sdim) // ROWS, N // CBLK),
        in_specs=[pl.BlockSpec((ROWS, CBLK), lambda i, j: (i, j))],
        out_specs=pl.BlockSpec((ROWS, TOPK), lambda i, j: (i, 0)),
        out_shape=jax.ShapeDtypeStruct((B * Sdim, TOPK), inputs.dtype),
        scratch_shapes=[pltpu.VMEM((ROWS, 8 * 128), jnp.float32)],
    )(x)
    return out.reshape(B, Sdim, TOPK)


# speedup vs baseline: 11.0483x; 1.8665x over previous
import jax
import jax.numpy as jnp
from jax.experimental import pallas as pl

TOPK = 8
ROWS = 512

_SORT8 = [
    (0, 1), (2, 3), (4, 5), (6, 7),
    (0, 2), (1, 3), (4, 6), (5, 7),
    (1, 2), (5, 6),
    (0, 4), (1, 5), (2, 6), (3, 7),
    (2, 4), (3, 5),
    (1, 2), (3, 4), (5, 6),
]

_BITONIC8 = [
    (0, 4), (1, 5), (2, 6), (3, 7),
    (0, 2), (1, 3), (4, 6), (5, 7),
    (0, 1), (2, 3), (4, 5), (6, 7),
]


def _apply_network(vals, network):
    for i, j in network:
        hi = jnp.maximum(vals[i], vals[j])
        lo = jnp.minimum(vals[i], vals[j])
        vals[i], vals[j] = hi, lo
    return vals


def _topk_block(x_ref, o_ref):
    R = x_ref.shape[0]
    neg = jnp.float32(-jnp.inf)

    S = None
    for g in range(8):
        grp = [x_ref[:, 128 * (8 * g + j):128 * (8 * g + j + 1)] for j in range(8)]
        grp = _apply_network(grp, _SORT8)
        if S is None:
            S = grp
        else:
            c = [jnp.maximum(S[i], grp[7 - i]) for i in range(8)]
            S = _apply_network(c, _BITONIC8)

    lane_iota = jax.lax.broadcasted_iota(jnp.int32, (R, 128), 1)
    outs = []
    for i in range(TOPK):
        m = jnp.max(S[0], axis=1, keepdims=True)
        outs.append(m)
        if i < TOPK - 1:
            li = jnp.where(S[0] == m, lane_iota, 128)
            first = jnp.min(li, axis=1, keepdims=True)
            mask = lane_iota == first
            depth = TOPK - 1 - i
            for j in range(depth):
                S[j] = jnp.where(mask, S[j + 1], S[j])
            S[depth] = jnp.where(mask, neg, S[depth])

    o_ref[...] = jnp.concatenate(outs, axis=1)


def kernel(inputs):
    B, Sdim, N = inputs.shape
    x = inputs.reshape(B * Sdim, N)
    out = pl.pallas_call(
        _topk_block,
        grid=((B * Sdim) // ROWS,),
        in_specs=[pl.BlockSpec((ROWS, N), lambda i: (i, 0))],
        out_specs=pl.BlockSpec((ROWS, TOPK), lambda i: (i, 0)),
        out_shape=jax.ShapeDtypeStruct((B * Sdim, TOPK), inputs.dtype),
    )(x)
    return out.reshape(B, Sdim, TOPK)


# speedup vs baseline: 11.1388x; 1.0082x over previous
import jax
import jax.numpy as jnp
from jax.experimental import pallas as pl
from jax.experimental.pallas import tpu as pltpu

TOPK = 8
ROWS = 512

_SORT8 = [
    (0, 1), (2, 3), (4, 5), (6, 7),
    (0, 2), (1, 3), (4, 6), (5, 7),
    (1, 2), (5, 6),
    (0, 4), (1, 5), (2, 6), (3, 7),
    (2, 4), (3, 5),
    (1, 2), (3, 4), (5, 6),
]

_BITONIC8 = [
    (0, 4), (1, 5), (2, 6), (3, 7),
    (0, 2), (1, 3), (4, 6), (5, 7),
    (0, 1), (2, 3), (4, 5), (6, 7),
]


def _apply_network(vals, network):
    for i, j in network:
        hi = jnp.maximum(vals[i], vals[j])
        lo = jnp.minimum(vals[i], vals[j])
        vals[i], vals[j] = hi, lo
    return vals


def _topk_block(x_ref, o_ref):
    R = x_ref.shape[0]
    neg = jnp.float32(-jnp.inf)

    S = None
    for g in range(8):
        grp = [x_ref[:, 128 * (8 * g + j):128 * (8 * g + j + 1)] for j in range(8)]
        grp = _apply_network(grp, _SORT8)
        if S is None:
            S = grp
        else:
            c = [jnp.maximum(S[i], grp[7 - i]) for i in range(8)]
            S = _apply_network(c, _BITONIC8)

    lane_iota = jax.lax.broadcasted_iota(jnp.int32, (R, 128), 1)
    outs = []
    for i in range(TOPK):
        m = jnp.max(S[0], axis=1, keepdims=True)
        outs.append(m)
        if i < TOPK - 1:
            li = jnp.where(S[0] == m, lane_iota, 128)
            first = jnp.min(li, axis=1, keepdims=True)
            mask = lane_iota == first
            depth = TOPK - 1 - i
            for j in range(depth):
                S[j] = jnp.where(mask, S[j + 1], S[j])
            S[depth] = jnp.where(mask, neg, S[depth])

    o_ref[...] = jnp.concatenate(outs, axis=1)


def kernel(inputs):
    B, Sdim, N = inputs.shape
    x = inputs.reshape(B * Sdim, N)
    out = pl.pallas_call(
        _topk_block,
        grid=((B * Sdim) // ROWS,),
        in_specs=[pl.BlockSpec((ROWS, N), lambda i: (i, 0))],
        out_specs=pl.BlockSpec((ROWS, TOPK), lambda i: (i, 0)),
        out_shape=jax.ShapeDtypeStruct((B * Sdim, TOPK), inputs.dtype),
        compiler_params=pltpu.CompilerParams(
            dimension_semantics=("parallel",)),
    )(x)
    return out.reshape(B, Sdim, TOPK)
